# Initial kernel scaffold; baseline (speedup 1.0000x reference)
import jax
import jax.numpy as jnp
from jax.experimental import pallas as pl


def kernel(h, mask, edge_index, W, b):
    raise NotImplementedError("write your pallas kernel here")


# speedup vs baseline: 17.7964x; 17.7964x over previous
import functools

import jax
import jax.numpy as jnp
from jax import lax
from jax.experimental import pallas as pl
from jax.experimental.pallas import tpu as pltpu
from jax.experimental.pallas import tpu_sc as plsc

N = 10000
E = 320000
D = 128

NC = 2
NS = 16
NW = NC * NS
EPW = E // NW
CHUNK = 80
NCHUNK = EPW // CHUNK
NPAD = 10240


def _sc_mesh():
    return plsc.VectorSubcoreMesh(core_axis_name="c", subcore_axis_name="s")


def _deg_body(e_src, e_dst, out, srcv, dstv, deg_s, deg_d):
    c = lax.axis_index("c")
    s = lax.axis_index("s")
    wid = c * NS + s
    pltpu.sync_copy(e_src.at[c, s], srcv)
    pltpu.sync_copy(e_dst.at[c, s], dstv)

    zeros16 = jnp.zeros((16,), jnp.float32)

    def zero_body(i, _):
        deg_s[pl.ds(i * 16, 16)] = zeros16
        deg_d[pl.ds(i * 16, 16)] = zeros16
        return _

    lax.fori_loop(0, NPAD // 16, zero_body, None)

    ones16 = jnp.ones((16,), jnp.float32)

    def hist_body(i, _):
        vs = srcv[pl.ds(i * 16, 16)]
        vd = dstv[pl.ds(i * 16, 16)]
        plsc.addupdate_scatter(deg_s, [vs], ones16)
        plsc.addupdate_scatter(deg_d, [vd], ones16)
        return _

    lax.fori_loop(0, EPW // 16, hist_body, None)

    pltpu.sync_copy(deg_s, out.at[wid, 0])
    pltpu.sync_copy(deg_d, out.at[wid, 1])


def _degrees(e_src_flat, e_dst_flat):
    k = pl.kernel(
        _deg_body,
        out_type=jax.ShapeDtypeStruct((NW, 2, NPAD), jnp.float32),
        mesh=_sc_mesh(),
        scratch_types=[
            pltpu.VMEM((EPW,), jnp.int32),
            pltpu.VMEM((EPW,), jnp.int32),
            pltpu.VMEM((NPAD,), jnp.float32),
            pltpu.VMEM((NPAD,), jnp.float32),
        ],
        compiler_params=pltpu.CompilerParams(needs_layout_passes=False),
    )
    return k(e_src_flat, e_dst_flat)


def _scales_body(degp_ref, maskf_ref, ssrc_ref, sdst_ref):
    deg = jnp.sum(degp_ref[...], axis=0)
    m = maskf_ref[...][:, 0][None, :]
    s = lax.rsqrt(jnp.maximum(deg, 1.0)) * m
    ssrc_ref[...] = s[0][:, None]
    sdst_ref[...] = s[1][:, None]


def _scales(degp, maskf):
    return pl.pallas_call(
        _scales_body,
        out_shape=[
            jax.ShapeDtypeStruct((NPAD, 1), jnp.float32),
            jax.ShapeDtypeStruct((NPAD, 1), jnp.float32),
        ],
    )(degp, maskf)


def _mul_body(h_ref, s_ref, g_ref):
    g_ref[...] = h_ref[...] * s_ref[...]


def _prescale(h, ssrc):
    rows = 400
    grid = N // rows
    return pl.pallas_call(
        _mul_body,
        grid=(grid,),
        in_specs=[
            pl.BlockSpec((rows, D), lambda i: (i, 0)),
            pl.BlockSpec((rows, 1), lambda i: (i, 0)),
        ],
        out_specs=pl.BlockSpec((rows, D), lambda i: (i, 0)),
        out_shape=jax.ShapeDtypeStruct((N, D), jnp.float32),
    )(h, ssrc)


DH = D // 2


def _agg_body(g0, g1, e_src, e_dst, out0, out1,
              srcv, dstv, buf, bounce, a_sh, sem):
    c = lax.axis_index("c")
    s = lax.axis_index("s")

    zeros16 = jnp.zeros((16,), jnp.float32)

    def zero_body(i, _):
        for kk in range(DH // 16):
            bounce[i, pl.ds(kk * 16, 16)] = zeros16
        return _

    pltpu.sync_copy(e_src.at[c, s], srcv)
    pltpu.sync_copy(e_dst.at[c, s], dstv)

    for g_hbm, out in ((g0, out0), (g1, out1)):
        lax.fori_loop(0, 128, zero_body, None)
        for kk in range(8):
            pltpu.sync_copy(bounce.at[pl.ds(0, CHUNK)],
                            a_sh.at[pl.ds(s * 640 + kk * CHUNK, CHUNK)])
        plsc.subcore_barrier()

        def edge_body(j, _):
            pltpu.async_copy(g_hbm.at[srcv.at[j]], buf, sem).wait()
            pltpu.sync_copy(buf, a_sh.at[dstv.at[j]], add=True)
            return _

        lax.fori_loop(0, NCHUNK, edge_body, None)
        plsc.subcore_barrier()

        for kk in range(5):
            base = s * 640 + kk * 128
            pltpu.sync_copy(a_sh.at[pl.ds(base, 128)], bounce)
            pltpu.sync_copy(bounce, out.at[c, pl.ds(base, 128)])
        plsc.subcore_barrier()


def _aggregate(g0, g1, e_src_chunk, e_dst_chunk):
    k = pl.kernel(
        _agg_body,
        out_type=[
            jax.ShapeDtypeStruct((NC, NPAD, DH), jnp.float32),
            jax.ShapeDtypeStruct((NC, NPAD, DH), jnp.float32),
        ],
        mesh=_sc_mesh(),
        scratch_types=[
            pltpu.VMEM((NCHUNK, CHUNK), jnp.int32),
            pltpu.VMEM((NCHUNK, CHUNK), jnp.int32),
            pltpu.VMEM((CHUNK, DH), jnp.float32),
            pltpu.VMEM((128, DH), jnp.float32),
            pltpu.VMEM_SHARED((NPAD, DH), jnp.float32),
            pltpu.SemaphoreType.DMA,
        ],
        compiler_params=pltpu.CompilerParams(use_tc_tiling_on_sc=False),
    )
    return k(g0, g1, e_src_chunk, e_dst_chunk)


def _out_body(p0_ref, p1_ref, sdst_ref, w0_ref, w1_ref, b_ref, o_ref):
    sd = sdst_ref[...]
    a0 = (p0_ref[0] + p0_ref[1]) * sd
    a1 = (p1_ref[0] + p1_ref[1]) * sd
    o_ref[...] = (
        jnp.dot(a0, w0_ref[...], preferred_element_type=jnp.float32)
        + jnp.dot(a1, w1_ref[...], preferred_element_type=jnp.float32)
        + b_ref[...]
    )


def _finalize(p0, p1, sdst, W, b2):
    rows = 400
    grid = N // rows
    return pl.pallas_call(
        _out_body,
        grid=(grid,),
        in_specs=[
            pl.BlockSpec((NC, rows, DH), lambda i: (0, i, 0)),
            pl.BlockSpec((NC, rows, DH), lambda i: (0, i, 0)),
            pl.BlockSpec((rows, 1), lambda i: (i, 0)),
            pl.BlockSpec((DH, D), lambda i: (0, 0)),
            pl.BlockSpec((DH, D), lambda i: (0, 0)),
            pl.BlockSpec((1, D), lambda i: (0, 0)),
        ],
        out_specs=pl.BlockSpec((rows, D), lambda i: (i, 0)),
        out_shape=jax.ShapeDtypeStruct((N, D), jnp.float32),
    )(p0, p1, sdst, W[:DH], W[DH:], b2)


def kernel(h, mask, edge_index, W, b):
    src = edge_index[0]
    dst = edge_index[1]
    e_src_flat = src.reshape(NC, NS, EPW)
    e_dst_flat = dst.reshape(NC, NS, EPW)
    e_src_chunk = src.reshape(NC, NS, NCHUNK, CHUNK)
    e_dst_chunk = dst.reshape(NC, NS, NCHUNK, CHUNK)
    maskf = jnp.pad(mask.astype(jnp.float32), (0, NPAD - N)).reshape(NPAD, 1)

    degp = _degrees(e_src_flat, e_dst_flat)
    ssrc, sdst = _scales(degp, maskf)
    g = _prescale(h, ssrc[:N])
    p0, p1 = _aggregate(g[:, :DH], g[:, DH:], e_src_chunk, e_dst_chunk)
    return _finalize(p0[:, :N], p1[:, :N], sdst[:N], W, b.reshape(1, D))


# speedup vs baseline: 22.1900x; 1.2469x over previous
import functools

import jax
import jax.numpy as jnp
from jax import lax
from jax.experimental import pallas as pl
from jax.experimental.pallas import tpu as pltpu
from jax.experimental.pallas import tpu_sc as plsc

N = 10000
E = 320000
D = 128

NC = 2
NS = 16
NW = NC * NS
EPW = E // NW
CHUNK = 80
NCHUNK = EPW // CHUNK
NPAD = 10240


def _sc_mesh():
    return plsc.VectorSubcoreMesh(core_axis_name="c", subcore_axis_name="s")


def _deg_body(e_src, e_dst, out, srcv, dstv, deg_s, deg_d):
    c = lax.axis_index("c")
    s = lax.axis_index("s")
    wid = c * NS + s
    pltpu.sync_copy(e_src.at[c, s], srcv)
    pltpu.sync_copy(e_dst.at[c, s], dstv)

    zeros16 = jnp.zeros((16,), jnp.float32)

    def zero_body(i, _):
        deg_s[pl.ds(i * 16, 16)] = zeros16
        deg_d[pl.ds(i * 16, 16)] = zeros16
        return _

    lax.fori_loop(0, NPAD // 16, zero_body, None)

    ones16 = jnp.ones((16,), jnp.float32)

    def hist_body(i, _):
        vs = srcv[pl.ds(i * 16, 16)]
        vd = dstv[pl.ds(i * 16, 16)]
        plsc.addupdate_scatter(deg_s, [vs], ones16)
        plsc.addupdate_scatter(deg_d, [vd], ones16)
        return _

    lax.fori_loop(0, EPW // 16, hist_body, None)

    pltpu.sync_copy(deg_s, out.at[wid, 0])
    pltpu.sync_copy(deg_d, out.at[wid, 1])


def _degrees(e_src_flat, e_dst_flat):
    k = pl.kernel(
        _deg_body,
        out_type=jax.ShapeDtypeStruct((NW, 2, NPAD), jnp.float32),
        mesh=_sc_mesh(),
        scratch_types=[
            pltpu.VMEM((EPW,), jnp.int32),
            pltpu.VMEM((EPW,), jnp.int32),
            pltpu.VMEM((NPAD,), jnp.float32),
            pltpu.VMEM((NPAD,), jnp.float32),
        ],
        compiler_params=pltpu.CompilerParams(needs_layout_passes=False),
    )
    return k(e_src_flat, e_dst_flat)


def _scales_body(degp_ref, maskf_ref, ssrc_ref, sdst_ref):
    deg = jnp.sum(degp_ref[...], axis=0)
    m = maskf_ref[...][:, 0][None, :]
    s = lax.rsqrt(jnp.maximum(deg, 1.0)) * m
    ssrc_ref[...] = s[0][:, None]
    sdst_ref[...] = s[1][:, None]


def _scales(degp, maskf):
    return pl.pallas_call(
        _scales_body,
        out_shape=[
            jax.ShapeDtypeStruct((NPAD, 1), jnp.float32),
            jax.ShapeDtypeStruct((NPAD, 1), jnp.float32),
        ],
    )(degp, maskf)


def _mul_body(h_ref, s_ref, g0_ref, g1_ref):
    g = h_ref[...] * s_ref[...]
    g0_ref[...] = g[:, :DH]
    g1_ref[...] = g[:, DH:]


def _prescale(h, ssrc):
    rows = 400
    grid = N // rows
    return pl.pallas_call(
        _mul_body,
        grid=(grid,),
        in_specs=[
            pl.BlockSpec((rows, D), lambda i: (i, 0)),
            pl.BlockSpec((rows, 1), lambda i: (i, 0)),
        ],
        out_specs=[
            pl.BlockSpec((rows, DH), lambda i: (i, 0)),
            pl.BlockSpec((rows, DH), lambda i: (i, 0)),
        ],
        out_shape=[
            jax.ShapeDtypeStruct((N, DH), jnp.float32),
            jax.ShapeDtypeStruct((N, DH), jnp.float32),
        ],
    )(h, ssrc)


DH = D // 2


def _agg_body(g0, g1, e_src, e_dst, out0, out1,
              srcv, dstv, bufa, bufb, bounce, a_sh, sem_a, sem_b):
    c = lax.axis_index("c")
    s = lax.axis_index("s")

    zeros16 = jnp.zeros((16,), jnp.float32)

    def zero_body(i, _):
        for kk in range(DH // 16):
            bounce[i, pl.ds(kk * 16, 16)] = zeros16
        return _

    pltpu.sync_copy(e_src.at[c, s], srcv)
    pltpu.sync_copy(e_dst.at[c, s], dstv)

    for g_hbm, out in ((g0, out0), (g1, out1)):
        lax.fori_loop(0, 128, zero_body, None)
        for kk in range(8):
            pltpu.sync_copy(bounce.at[pl.ds(0, CHUNK)],
                            a_sh.at[pl.ds(s * 640 + kk * CHUNK, CHUNK)])
        plsc.subcore_barrier()

        pltpu.async_copy(g_hbm.at[srcv.at[0]], bufa, sem_a)

        def edge_pair(j, _):
            a = 2 * j
            pltpu.make_async_copy(g_hbm.at[srcv.at[0]], bufa, sem_a).wait()
            pltpu.async_copy(g_hbm.at[srcv.at[a + 1]], bufb, sem_b)
            pltpu.sync_copy(bufa, a_sh.at[dstv.at[a]], add=True)
            pltpu.make_async_copy(g_hbm.at[srcv.at[0]], bufb, sem_b).wait()

            @pl.when(a + 2 < NCHUNK)
            def _():
                pltpu.async_copy(g_hbm.at[srcv.at[a + 2]], bufa, sem_a)

            pltpu.sync_copy(bufb, a_sh.at[dstv.at[a + 1]], add=True)
            return _

        lax.fori_loop(0, NCHUNK // 2, edge_pair, None)
        pltpu.make_async_copy(g_hbm.at[srcv.at[0]], bufa, sem_a).wait()
        pltpu.sync_copy(bufa, a_sh.at[dstv.at[NCHUNK - 1]], add=True)
        plsc.subcore_barrier()

        for kk in range(5):
            base = s * 640 + kk * 128
            pltpu.sync_copy(a_sh.at[pl.ds(base, 128)], bounce)
            pltpu.sync_copy(bounce, out.at[c, pl.ds(base, 128)])
        plsc.subcore_barrier()


def _aggregate(g0, g1, e_src_chunk, e_dst_chunk):
    k = pl.kernel(
        _agg_body,
        out_type=[
            jax.ShapeDtypeStruct((NC, NPAD, DH), jnp.float32),
            jax.ShapeDtypeStruct((NC, NPAD, DH), jnp.float32),
        ],
        mesh=_sc_mesh(),
        scratch_types=[
            pltpu.VMEM((NCHUNK, CHUNK), jnp.int32),
            pltpu.VMEM((NCHUNK, CHUNK), jnp.int32),
            pltpu.VMEM((CHUNK, DH), jnp.float32),
            pltpu.VMEM((CHUNK, DH), jnp.float32),
            pltpu.VMEM((128, DH), jnp.float32),
            pltpu.VMEM_SHARED((NPAD, DH), jnp.float32),
            pltpu.SemaphoreType.DMA,
            pltpu.SemaphoreType.DMA,
        ],
        compiler_params=pltpu.CompilerParams(use_tc_tiling_on_sc=False),
    )
    return k(g0, g1, e_src_chunk, e_dst_chunk)


def _out_body(p0_ref, p1_ref, sdst_ref, w0_ref, w1_ref, b_ref, o_ref):
    sd = sdst_ref[...]
    a0 = (p0_ref[0] + p0_ref[1]) * sd
    a1 = (p1_ref[0] + p1_ref[1]) * sd
    o_ref[...] = (
        jnp.dot(a0, w0_ref[...], preferred_element_type=jnp.float32)
        + jnp.dot(a1, w1_ref[...], preferred_element_type=jnp.float32)
        + b_ref[...]
    )


def _finalize(p0, p1, sdst, W, b2):
    rows = 400
    grid = N // rows
    return pl.pallas_call(
        _out_body,
        grid=(grid,),
        in_specs=[
            pl.BlockSpec((NC, rows, DH), lambda i: (0, i, 0)),
            pl.BlockSpec((NC, rows, DH), lambda i: (0, i, 0)),
            pl.BlockSpec((rows, 1), lambda i: (i, 0)),
            pl.BlockSpec((DH, D), lambda i: (0, 0)),
            pl.BlockSpec((DH, D), lambda i: (0, 0)),
            pl.BlockSpec((1, D), lambda i: (0, 0)),
        ],
        out_specs=pl.BlockSpec((rows, D), lambda i: (i, 0)),
        out_shape=jax.ShapeDtypeStruct((N, D), jnp.float32),
    )(p0, p1, sdst, W[:DH], W[DH:], b2)


def kernel(h, mask, edge_index, W, b):
    src = edge_index[0]
    dst = edge_index[1]
    e_src_flat = src.reshape(NC, NS, EPW)
    e_dst_flat = dst.reshape(NC, NS, EPW)
    e_src_chunk = src.reshape(NC, NS, NCHUNK, CHUNK)
    e_dst_chunk = dst.reshape(NC, NS, NCHUNK, CHUNK)
    maskf = jnp.pad(mask.astype(jnp.float32), (0, NPAD - N)).reshape(NPAD, 1)

    degp = _degrees(e_src_flat, e_dst_flat)
    ssrc, sdst = _scales(degp, maskf)
    g0, g1 = _prescale(h, ssrc)
    p0, p1 = _aggregate(g0, g1, e_src_chunk, e_dst_chunk)
    return _finalize(p0, p1, sdst, W, b.reshape(1, D))


# speedup vs baseline: 29.8194x; 1.3438x over previous
import functools

import jax
import jax.numpy as jnp
from jax import lax
from jax.experimental import pallas as pl
from jax.experimental.pallas import tpu as pltpu
from jax.experimental.pallas import tpu_sc as plsc

N = 10000
E = 320000
D = 128

NC = 2
NS = 16
NW = NC * NS
EPW = E // NW
CHUNK = 80
NCHUNK = EPW // CHUNK
NPAD = 10240


def _sc_mesh():
    return plsc.VectorSubcoreMesh(core_axis_name="c", subcore_axis_name="s")


FCAP = 10080


def _deg_body(e_src, e_dst, maski, out, fidx, cnt,
              srcv, dstv, maskv, deg_s, deg_d, fsrcv, fdstv, cntv):
    c = lax.axis_index("c")
    s = lax.axis_index("s")
    wid = c * NS + s
    pltpu.sync_copy(e_src.at[c, s], srcv)
    pltpu.sync_copy(e_dst.at[c, s], dstv)
    pltpu.sync_copy(maski, maskv)

    zeros16 = jnp.zeros((16,), jnp.float32)
    iota16 = lax.iota(jnp.int32, 16)

    def zero_body(i, _):
        deg_s[pl.ds(i * 16, 16)] = zeros16
        deg_d[pl.ds(i * 16, 16)] = zeros16
        return _

    lax.fori_loop(0, NPAD // 16, zero_body, None)

    def fill_body(i, _):
        base = i * 16
        fsrcv[pl.ds(base, 16)] = ((wid * 613 + base) % 9984) + iota16
        fdstv[pl.ds(base, 16)] = N + ((base + wid) % 224) + iota16
        return _

    lax.fori_loop(0, FCAP // 16, fill_body, None)

    ones16 = jnp.ones((16,), jnp.float32)

    def hist_body(i, off):
        vs = srcv[pl.ds(i * 16, 16)]
        vd = dstv[pl.ds(i * 16, 16)]
        plsc.addupdate_scatter(deg_s, [vs], ones16)
        plsc.addupdate_scatter(deg_d, [vd], ones16)
        ms = plsc.load_gather(maskv, [vs])
        md = plsc.load_gather(maskv, [vd])
        act = (ms & md) != 0
        plsc.store_compressed(fsrcv.at[pl.ds(off, 16)], vs, mask=act)
        plsc.store_compressed(fdstv.at[pl.ds(off, 16)], vd, mask=act)
        npc = plsc.all_reduce_population_count(act)
        return off + lax.reduce_max(npc, (0,))

    off = lax.fori_loop(0, EPW // 16, hist_body, 0)

    end = ((off + 2 * FC - 1) // (2 * FC)) * (2 * FC)

    def pad_body(i, _):
        base = off + i * 16
        fsrcv[pl.ds(base, 16)] = ((wid * 613 + base) % 9984) + iota16
        fdstv[pl.ds(base, 16)] = N + ((base + wid) % 224) + iota16
        return _

    lax.fori_loop(0, (end - off) // 16, pad_body, None)

    cntv[pl.ds(0, 16)] = jnp.broadcast_to(end // FC, (16,))

    plsc.subcore_barrier()

    pltpu.sync_copy(deg_s, out.at[wid, 0])
    pltpu.sync_copy(deg_d, out.at[wid, 1])
    pltpu.sync_copy(fsrcv, fidx.at[wid, 0])
    pltpu.sync_copy(fdstv, fidx.at[wid, 1])
    pltpu.sync_copy(cntv, cnt.at[wid, 0])


def _degrees(e_src_flat, e_dst_flat, maski):
    k = pl.kernel(
        _deg_body,
        out_type=[
            jax.ShapeDtypeStruct((NW, 2, NPAD), jnp.float32),
            jax.ShapeDtypeStruct((NW, 2, FCAP), jnp.int32),
            jax.ShapeDtypeStruct((NW, 1, 16), jnp.int32),
        ],
        mesh=_sc_mesh(),
        scratch_types=[
            pltpu.VMEM((EPW,), jnp.int32),
            pltpu.VMEM((EPW,), jnp.int32),
            pltpu.VMEM((NPAD,), jnp.int32),
            pltpu.VMEM((NPAD,), jnp.float32),
            pltpu.VMEM((NPAD,), jnp.float32),
            pltpu.VMEM((FCAP,), jnp.int32),
            pltpu.VMEM((FCAP,), jnp.int32),
            pltpu.VMEM((16,), jnp.int32),
        ],
        compiler_params=pltpu.CompilerParams(needs_layout_passes=False),
    )
    return k(e_src_flat, e_dst_flat, maski)


def _scales_body(degp_ref, maskf_ref, ssrc_ref, sdst_ref):
    deg = jnp.sum(degp_ref[...], axis=0)
    m = maskf_ref[...][:, 0][None, :]
    s = lax.rsqrt(jnp.maximum(deg, 1.0)) * m
    ssrc_ref[...] = s[0][:, None]
    sdst_ref[...] = s[1][:, None]


def _scales(degp, maskf):
    return pl.pallas_call(
        _scales_body,
        out_shape=[
            jax.ShapeDtypeStruct((NPAD, 1), jnp.float32),
            jax.ShapeDtypeStruct((NPAD, 1), jnp.float32),
        ],
    )(degp, maskf)


def _mul_body(h_ref, s_ref, g0_ref, g1_ref):
    g = h_ref[...] * s_ref[...]
    g0_ref[...] = g[:, :DH]
    g1_ref[...] = g[:, DH:]


def _prescale(h, ssrc):
    rows = 400
    grid = N // rows
    return pl.pallas_call(
        _mul_body,
        grid=(grid,),
        in_specs=[
            pl.BlockSpec((rows, D), lambda i: (i, 0)),
            pl.BlockSpec((rows, 1), lambda i: (i, 0)),
        ],
        out_specs=[
            pl.BlockSpec((rows, DH), lambda i: (i, 0)),
            pl.BlockSpec((rows, DH), lambda i: (i, 0)),
        ],
        out_shape=[
            jax.ShapeDtypeStruct((N, DH), jnp.float32),
            jax.ShapeDtypeStruct((N, DH), jnp.float32),
        ],
    )(h, ssrc)


def _relayout_idx(x):
    nfc, fc = FCAP // 80, 80

    def body(i_ref, s_ref, d_ref):
        s_ref[...] = i_ref[0, 0].reshape(1, 1, nfc, fc)
        d_ref[...] = i_ref[0, 1].reshape(1, 1, nfc, fc)

    return pl.pallas_call(
        body,
        grid=(NW,),
        in_specs=[pl.BlockSpec((1, 2, FCAP), lambda i: (i, 0, 0))],
        out_specs=[
            pl.BlockSpec((1, 1, nfc, fc), lambda i: (i // NS, i % NS, 0, 0)),
            pl.BlockSpec((1, 1, nfc, fc), lambda i: (i // NS, i % NS, 0, 0)),
        ],
        out_shape=[
            jax.ShapeDtypeStruct((NC, NS, nfc, fc), jnp.int32),
            jax.ShapeDtypeStruct((NC, NS, nfc, fc), jnp.int32),
        ],
    )(x)


def _relayout_cnt(x):
    def body(i_ref, o_ref):
        o_ref[...] = i_ref[...].reshape(NC, NS, 1, 16)

    return pl.pallas_call(
        body,
        out_shape=jax.ShapeDtypeStruct((NC, NS, 1, 16), jnp.int32),
    )(x)


DH = D // 2
FC = 80
NFC = FCAP // FC


def _agg_body(g0, g1, e_src, e_dst, cnt, out0, out1,
              srcv, dstv, cntv, bufa, bufb, bounce, a_sh, sem_a, sem_b):
    c = lax.axis_index("c")
    s = lax.axis_index("s")

    zeros16 = jnp.zeros((16,), jnp.float32)

    def zero_body(i, _):
        for kk in range(DH // 16):
            bounce[i, pl.ds(kk * 16, 16)] = zeros16
        return _

    pltpu.sync_copy(e_src.at[c, s], srcv)
    pltpu.sync_copy(e_dst.at[c, s], dstv)
    pltpu.sync_copy(cnt.at[c, s], cntv)
    nch = lax.reduce_max(cntv[0, pl.ds(0, 16)], (0,))
    nch = jnp.minimum(jnp.maximum(nch, 0), NFC)

    for g_hbm, out in ((g0, out0), (g1, out1)):
        lax.fori_loop(0, 128, zero_body, None)
        for kk in range(8):
            pltpu.sync_copy(bounce.at[pl.ds(0, 80)],
                            a_sh.at[pl.ds(s * 640 + kk * 80, 80)])
        plsc.subcore_barrier()

        def edge_body(j, _):
            pltpu.sync_copy(g_hbm.at[srcv.at[j]], bufa)
            pltpu.sync_copy(bufa, a_sh.at[dstv.at[j]], add=True)
            return _

        lax.fori_loop(0, nch, edge_body, None)
        plsc.subcore_barrier()

        for kk in range(5):
            base = s * 640 + kk * 128
            pltpu.sync_copy(a_sh.at[pl.ds(base, 128)], bounce)
            pltpu.sync_copy(bounce, out.at[c, pl.ds(base, 128)])
        plsc.subcore_barrier()


def _aggregate(g0, g1, e_src_chunk, e_dst_chunk, cnt_chunk):
    k = pl.kernel(
        _agg_body,
        out_type=[
            jax.ShapeDtypeStruct((NC, NPAD, DH), jnp.float32),
            jax.ShapeDtypeStruct((NC, NPAD, DH), jnp.float32),
        ],
        mesh=_sc_mesh(),
        scratch_types=[
            pltpu.VMEM((NFC, FC), jnp.int32),
            pltpu.VMEM((NFC, FC), jnp.int32),
            pltpu.VMEM((1, 16), jnp.int32),
            pltpu.VMEM((FC, DH), jnp.float32),
            pltpu.VMEM((FC, DH), jnp.float32),
            pltpu.VMEM((128, DH), jnp.float32),
            pltpu.VMEM_SHARED((NPAD, DH), jnp.float32),
            pltpu.SemaphoreType.DMA,
            pltpu.SemaphoreType.DMA,
        ],
        compiler_params=pltpu.CompilerParams(
            use_tc_tiling_on_sc=False, needs_layout_passes=False),
    )
    return k(g0, g1, e_src_chunk, e_dst_chunk, cnt_chunk)


def _out_body(p0_ref, p1_ref, sdst_ref, w0_ref, w1_ref, b_ref, o_ref):
    sd = sdst_ref[...]
    a0 = (p0_ref[0] + p0_ref[1]) * sd
    a1 = (p1_ref[0] + p1_ref[1]) * sd
    o_ref[...] = (
        jnp.dot(a0, w0_ref[...], preferred_element_type=jnp.float32)
        + jnp.dot(a1, w1_ref[...], preferred_element_type=jnp.float32)
        + b_ref[...]
    )


def _finalize(p0, p1, sdst, W, b2):
    rows = 400
    grid = N // rows
    return pl.pallas_call(
        _out_body,
        grid=(grid,),
        in_specs=[
            pl.BlockSpec((NC, rows, DH), lambda i: (0, i, 0)),
            pl.BlockSpec((NC, rows, DH), lambda i: (0, i, 0)),
            pl.BlockSpec((rows, 1), lambda i: (i, 0)),
            pl.BlockSpec((DH, D), lambda i: (0, 0)),
            pl.BlockSpec((DH, D), lambda i: (0, 0)),
            pl.BlockSpec((1, D), lambda i: (0, 0)),
        ],
        out_specs=pl.BlockSpec((rows, D), lambda i: (i, 0)),
        out_shape=jax.ShapeDtypeStruct((N, D), jnp.float32),
    )(p0, p1, sdst, W[:DH], W[DH:], b2)


def kernel(h, mask, edge_index, W, b):
    src = edge_index[0]
    dst = edge_index[1]
    e_src_flat = src.reshape(NC, NS, EPW)
    e_dst_flat = dst.reshape(NC, NS, EPW)
    maski = jnp.pad(mask.astype(jnp.int32), (0, NPAD - N))
    maskf = jnp.pad(mask.astype(jnp.float32), (0, NPAD - N)).reshape(NPAD, 1)

    degp, fidx, cnt = _degrees(e_src_flat, e_dst_flat, maski)
    ssrc, sdst = _scales(degp, maskf)
    g0, g1 = _prescale(h, ssrc)
    fsrc4, fdst4 = _relayout_idx(fidx)
    p0, p1 = _aggregate(g0, g1, fsrc4, fdst4, _relayout_cnt(cnt))
    return _finalize(p0, p1, sdst, W, b.reshape(1, D))


# speedup vs baseline: 34.2517x; 1.1486x over previous
import functools

import jax
import jax.numpy as jnp
from jax import lax
from jax.experimental import pallas as pl
from jax.experimental.pallas import tpu as pltpu
from jax.experimental.pallas import tpu_sc as plsc

N = 10000
E = 320000
D = 128

NC = 2
NS = 16
NW = NC * NS
EPW = E // NW
CHUNK = 80
NCHUNK = EPW // CHUNK
NPAD = 10240


def _sc_mesh():
    return plsc.VectorSubcoreMesh(core_axis_name="c", subcore_axis_name="s")


FCAP = 10080


def _deg_body(e_src, e_dst, maski, out, fidx, cnt,
              srcv, dstv, maskv, deg_s, deg_d, fsrcv, fdstv, cntv):
    c = lax.axis_index("c")
    s = lax.axis_index("s")
    wid = c * NS + s
    pltpu.sync_copy(e_src.at[c, s], srcv)
    pltpu.sync_copy(e_dst.at[c, s], dstv)
    pltpu.sync_copy(maski, maskv)

    zeros16 = jnp.zeros((16,), jnp.float32)
    iota16 = lax.iota(jnp.int32, 16)

    def zero_body(i, _):
        deg_s[pl.ds(i * 16, 16)] = zeros16
        deg_d[pl.ds(i * 16, 16)] = zeros16
        return _

    lax.fori_loop(0, NPAD // 16, zero_body, None)

    def fill_body(i, _):
        base = i * 16
        fsrcv[pl.ds(base, 16)] = ((wid * 613 + base) % 9984) + iota16
        fdstv[pl.ds(base, 16)] = N + ((base + wid) % 224) + iota16
        return _

    lax.fori_loop(0, FCAP // 16, fill_body, None)

    ones16 = jnp.ones((16,), jnp.float32)

    def hist_body(i, off):
        vs = srcv[pl.ds(i * 16, 16)]
        vd = dstv[pl.ds(i * 16, 16)]
        plsc.addupdate_scatter(deg_s, [vs], ones16)
        plsc.addupdate_scatter(deg_d, [vd], ones16)
        ms = plsc.load_gather(maskv, [vs])
        md = plsc.load_gather(maskv, [vd])
        act = (ms & md) != 0
        plsc.store_compressed(fsrcv.at[pl.ds(off, 16)], vs, mask=act)
        plsc.store_compressed(fdstv.at[pl.ds(off, 16)], vd, mask=act)
        npc = plsc.all_reduce_population_count(act)
        return off + lax.reduce_max(npc, (0,))

    off = lax.fori_loop(0, EPW // 16, hist_body, 0)

    end = ((off + 2 * FC - 1) // (2 * FC)) * (2 * FC)

    def pad_body(i, _):
        base = off + i * 16
        fsrcv[pl.ds(base, 16)] = ((wid * 613 + base) % 9984) + iota16
        fdstv[pl.ds(base, 16)] = N + ((base + wid) % 224) + iota16
        return _

    lax.fori_loop(0, (end - off) // 16, pad_body, None)

    cntv[pl.ds(0, 16)] = jnp.broadcast_to(end // FC, (16,))

    plsc.subcore_barrier()

    pltpu.sync_copy(deg_s, out.at[wid, 0])
    pltpu.sync_copy(deg_d, out.at[wid, 1])
    pltpu.sync_copy(fsrcv, fidx.at[wid, 0])
    pltpu.sync_copy(fdstv, fidx.at[wid, 1])
    pltpu.sync_copy(cntv, cnt.at[wid, 0])


def _degrees(e_src_flat, e_dst_flat, maski):
    k = pl.kernel(
        _deg_body,
        out_type=[
            jax.ShapeDtypeStruct((NW, 2, NPAD), jnp.float32),
            jax.ShapeDtypeStruct((NW, 2, FCAP), jnp.int32),
            jax.ShapeDtypeStruct((NW, 1, 16), jnp.int32),
        ],
        mesh=_sc_mesh(),
        scratch_types=[
            pltpu.VMEM((EPW,), jnp.int32),
            pltpu.VMEM((EPW,), jnp.int32),
            pltpu.VMEM((NPAD,), jnp.int32),
            pltpu.VMEM((NPAD,), jnp.float32),
            pltpu.VMEM((NPAD,), jnp.float32),
            pltpu.VMEM((FCAP,), jnp.int32),
            pltpu.VMEM((FCAP,), jnp.int32),
            pltpu.VMEM((16,), jnp.int32),
        ],
        compiler_params=pltpu.CompilerParams(needs_layout_passes=False),
    )
    return k(e_src_flat, e_dst_flat, maski)


def _scales_body(degp_ref, maskf_ref, ssrc_ref, sdst_ref):
    deg = jnp.sum(degp_ref[...], axis=0)
    m = maskf_ref[...][:, 0][None, :]
    s = lax.rsqrt(jnp.maximum(deg, 1.0)) * m
    ssrc_ref[...] = s[0][:, None]
    sdst_ref[...] = s[1][:, None]


def _scales(degp, maskf):
    return pl.pallas_call(
        _scales_body,
        out_shape=[
            jax.ShapeDtypeStruct((NPAD, 1), jnp.float32),
            jax.ShapeDtypeStruct((NPAD, 1), jnp.float32),
        ],
    )(degp, maskf)


def _mul_body(h_ref, s_ref, g0_ref, g1_ref):
    g = h_ref[...] * s_ref[...]
    g0_ref[...] = g[:, :DH]
    g1_ref[...] = g[:, DH:]


def _prescale(h, ssrc):
    rows = 400
    grid = N // rows
    return pl.pallas_call(
        _mul_body,
        grid=(grid,),
        in_specs=[
            pl.BlockSpec((rows, D), lambda i: (i, 0)),
            pl.BlockSpec((rows, 1), lambda i: (i, 0)),
        ],
        out_specs=[
            pl.BlockSpec((rows, DH), lambda i: (i, 0)),
            pl.BlockSpec((rows, DH), lambda i: (i, 0)),
        ],
        out_shape=[
            jax.ShapeDtypeStruct((N, DH), jnp.float32),
            jax.ShapeDtypeStruct((N, DH), jnp.float32),
        ],
    )(h, ssrc)


def _relayout_idx(x):
    nfc, fc = FCAP // 80, 80

    def body(i_ref, s_ref, d_ref):
        s_ref[...] = i_ref[0, 0].reshape(1, 1, nfc, fc)
        d_ref[...] = i_ref[0, 1].reshape(1, 1, nfc, fc)

    return pl.pallas_call(
        body,
        grid=(NW,),
        in_specs=[pl.BlockSpec((1, 2, FCAP), lambda i: (i, 0, 0))],
        out_specs=[
            pl.BlockSpec((1, 1, nfc, fc), lambda i: (i // NS, i % NS, 0, 0)),
            pl.BlockSpec((1, 1, nfc, fc), lambda i: (i // NS, i % NS, 0, 0)),
        ],
        out_shape=[
            jax.ShapeDtypeStruct((NC, NS, nfc, fc), jnp.int32),
            jax.ShapeDtypeStruct((NC, NS, nfc, fc), jnp.int32),
        ],
    )(x)


def _relayout_cnt(x):
    def body(i_ref, o_ref):
        o_ref[...] = i_ref[...].reshape(NC, NS, 1, 16)

    return pl.pallas_call(
        body,
        out_shape=jax.ShapeDtypeStruct((NC, NS, 1, 16), jnp.int32),
    )(x)


DH = D // 2
FC = 80
NFC = FCAP // FC


def _agg_body(g0, g1, e_src, e_dst, cnt, out0, out1,
              srcv, dstv, cntv, bufa, bufb, bounce, a_sh, sem_a, sem_b):
    c = lax.axis_index("c")
    s = lax.axis_index("s")

    zeros16 = jnp.zeros((16,), jnp.float32)

    def zero_body(i, _):
        for kk in range(DH // 16):
            bounce[i, pl.ds(kk * 16, 16)] = zeros16
        return _

    pltpu.sync_copy(e_src.at[c, s], srcv)
    pltpu.sync_copy(e_dst.at[c, s], dstv)
    pltpu.sync_copy(cnt.at[c, s], cntv)
    nch = lax.reduce_max(cntv[0, pl.ds(0, 16)], (0,))
    nch = jnp.minimum(jnp.maximum(nch, 0), NFC)

    for g_hbm, out in ((g0, out0), (g1, out1)):
        lax.fori_loop(0, 128, zero_body, None)
        for kk in range(8):
            pltpu.sync_copy(bounce.at[pl.ds(0, 80)],
                            a_sh.at[pl.ds(s * 640 + kk * 80, 80)])
        plsc.subcore_barrier()

        @pl.when(nch > 0)
        def _():
            pltpu.async_copy(g_hbm.at[srcv.at[0]], bufa, sem_a)

        def edge_pair(j, _):
            a = 2 * j
            pltpu.make_async_copy(g_hbm.at[srcv.at[0]], bufa, sem_a).wait()
            pltpu.async_copy(g_hbm.at[srcv.at[a + 1]], bufb, sem_b)
            pltpu.sync_copy(bufa, a_sh.at[dstv.at[a]], add=True)
            pltpu.make_async_copy(g_hbm.at[srcv.at[0]], bufb, sem_b).wait()

            @pl.when(a + 2 < nch)
            def _():
                pltpu.async_copy(g_hbm.at[srcv.at[a + 2]], bufa, sem_a)

            pltpu.sync_copy(bufb, a_sh.at[dstv.at[a + 1]], add=True)
            return _

        lax.fori_loop(0, nch // 2, edge_pair, None)
        plsc.subcore_barrier()

        for kk in range(5):
            base = s * 640 + kk * 128
            pltpu.sync_copy(a_sh.at[pl.ds(base, 128)], bounce)
            pltpu.sync_copy(bounce, out.at[c, pl.ds(base, 128)])
        plsc.subcore_barrier()


def _aggregate(g0, g1, e_src_chunk, e_dst_chunk, cnt_chunk):
    k = pl.kernel(
        _agg_body,
        out_type=[
            jax.ShapeDtypeStruct((NC, NPAD, DH), jnp.float32),
            jax.ShapeDtypeStruct((NC, NPAD, DH), jnp.float32),
        ],
        mesh=_sc_mesh(),
        scratch_types=[
            pltpu.VMEM((NFC, FC), jnp.int32),
            pltpu.VMEM((NFC, FC), jnp.int32),
            pltpu.VMEM((1, 16), jnp.int32),
            pltpu.VMEM((FC, DH), jnp.float32),
            pltpu.VMEM((FC, DH), jnp.float32),
            pltpu.VMEM((128, DH), jnp.float32),
            pltpu.VMEM_SHARED((NPAD, DH), jnp.float32),
            pltpu.SemaphoreType.DMA,
            pltpu.SemaphoreType.DMA,
        ],
        compiler_params=pltpu.CompilerParams(
            use_tc_tiling_on_sc=False, needs_layout_passes=False),
    )
    return k(g0, g1, e_src_chunk, e_dst_chunk, cnt_chunk)


def _out_body(p0_ref, p1_ref, sdst_ref, w0_ref, w1_ref, b_ref, o_ref):
    sd = sdst_ref[...]
    a0 = (p0_ref[0] + p0_ref[1]) * sd
    a1 = (p1_ref[0] + p1_ref[1]) * sd
    o_ref[...] = (
        jnp.dot(a0, w0_ref[...], preferred_element_type=jnp.float32)
        + jnp.dot(a1, w1_ref[...], preferred_element_type=jnp.float32)
        + b_ref[...]
    )


def _finalize(p0, p1, sdst, W, b2):
    rows = 400
    grid = N // rows
    return pl.pallas_call(
        _out_body,
        grid=(grid,),
        in_specs=[
            pl.BlockSpec((NC, rows, DH), lambda i: (0, i, 0)),
            pl.BlockSpec((NC, rows, DH), lambda i: (0, i, 0)),
            pl.BlockSpec((rows, 1), lambda i: (i, 0)),
            pl.BlockSpec((DH, D), lambda i: (0, 0)),
            pl.BlockSpec((DH, D), lambda i: (0, 0)),
            pl.BlockSpec((1, D), lambda i: (0, 0)),
        ],
        out_specs=pl.BlockSpec((rows, D), lambda i: (i, 0)),
        out_shape=jax.ShapeDtypeStruct((N, D), jnp.float32),
    )(p0, p1, sdst, W[:DH], W[DH:], b2)


def kernel(h, mask, edge_index, W, b):
    src = edge_index[0]
    dst = edge_index[1]
    e_src_flat = src.reshape(NC, NS, EPW)
    e_dst_flat = dst.reshape(NC, NS, EPW)
    maski = jnp.pad(mask.astype(jnp.int32), (0, NPAD - N))
    maskf = jnp.pad(mask.astype(jnp.float32), (0, NPAD - N)).reshape(NPAD, 1)

    degp, fidx, cnt = _degrees(e_src_flat, e_dst_flat, maski)
    ssrc, sdst = _scales(degp, maskf)
    g0, g1 = _prescale(h, ssrc)
    p0, p1 = _aggregate(g0, g1,
                        fidx[:, 0].reshape(NC, NS, NFC, FC),
                        fidx[:, 1].reshape(NC, NS, NFC, FC),
                        cnt.reshape(NC, NS, 1, 16))
    return _finalize(p0, p1, sdst, W, b.reshape(1, D))


# speedup vs baseline: 34.2806x; 1.0008x over previous
import jax
import jax.numpy as jnp
from jax import lax
from jax.experimental import pallas as pl
from jax.experimental.pallas import tpu as pltpu
from jax.experimental.pallas import tpu_sc as plsc

N = 10000
E = 320000
D = 128

NC = 2
NS = 16
NW = NC * NS
EPW = E // NW
NPAD = 10240


def _sc_mesh():
    return plsc.VectorSubcoreMesh(core_axis_name="c", subcore_axis_name="s")


FCAP = 10080


def _deg_body(e_src, e_dst, maski, out, fidx, cnt,
              srcv, dstv, maskv, deg_s, deg_d, fsrcv, fdstv, cntv):
    c = lax.axis_index("c")
    s = lax.axis_index("s")
    wid = c * NS + s
    pltpu.sync_copy(e_src.at[c, s], srcv)
    pltpu.sync_copy(e_dst.at[c, s], dstv)
    pltpu.sync_copy(maski, maskv)

    zeros16 = jnp.zeros((16,), jnp.float32)
    iota16 = lax.iota(jnp.int32, 16)

    def zero_body(i, _):
        deg_s[pl.ds(i * 16, 16)] = zeros16
        deg_d[pl.ds(i * 16, 16)] = zeros16
        return _

    lax.fori_loop(0, NPAD // 16, zero_body, None)

    def fill_body(i, _):
        base = i * 16
        fsrcv[pl.ds(base, 16)] = ((wid * 613 + base) % 9984) + iota16
        fdstv[pl.ds(base, 16)] = N + ((base + wid) % 224) + iota16
        return _

    lax.fori_loop(0, FCAP // 16, fill_body, None)

    ones16 = jnp.ones((16,), jnp.float32)

    def hist_body(i, off):
        vs = srcv[pl.ds(i * 16, 16)]
        vd = dstv[pl.ds(i * 16, 16)]
        plsc.addupdate_scatter(deg_s, [vs], ones16)
        plsc.addupdate_scatter(deg_d, [vd], ones16)
        ms = plsc.load_gather(maskv, [vs])
        md = plsc.load_gather(maskv, [vd])
        act = (ms & md) != 0
        plsc.store_compressed(fsrcv.at[pl.ds(off, 16)], vs, mask=act)
        plsc.store_compressed(fdstv.at[pl.ds(off, 16)], vd, mask=act)
        npc = plsc.all_reduce_population_count(act)
        return off + lax.reduce_max(npc, (0,))

    off = lax.fori_loop(0, EPW // 16, hist_body, 0)

    end = ((off + 2 * FC - 1) // (2 * FC)) * (2 * FC)

    def pad_body(i, _):
        base = off + i * 16
        fsrcv[pl.ds(base, 16)] = ((wid * 613 + base) % 9984) + iota16
        fdstv[pl.ds(base, 16)] = N + ((base + wid) % 224) + iota16
        return _

    lax.fori_loop(0, (end - off) // 16, pad_body, None)

    cntv[pl.ds(0, 16)] = jnp.broadcast_to(end // FC, (16,))

    plsc.subcore_barrier()

    pltpu.sync_copy(deg_s, out.at[wid, 0])
    pltpu.sync_copy(deg_d, out.at[wid, 1])
    pltpu.sync_copy(fsrcv, fidx.at[wid, 0])
    pltpu.sync_copy(fdstv, fidx.at[wid, 1])
    pltpu.sync_copy(cntv, cnt.at[wid, 0])


def _degrees(e_src_flat, e_dst_flat, maski):
    k = pl.kernel(
        _deg_body,
        out_type=[
            jax.ShapeDtypeStruct((NW, 2, NPAD), jnp.float32),
            jax.ShapeDtypeStruct((NW, 2, FCAP), jnp.int32),
            jax.ShapeDtypeStruct((NW, 1, 16), jnp.int32),
        ],
        mesh=_sc_mesh(),
        scratch_types=[
            pltpu.VMEM((EPW,), jnp.int32),
            pltpu.VMEM((EPW,), jnp.int32),
            pltpu.VMEM((NPAD,), jnp.int32),
            pltpu.VMEM((NPAD,), jnp.float32),
            pltpu.VMEM((NPAD,), jnp.float32),
            pltpu.VMEM((FCAP,), jnp.int32),
            pltpu.VMEM((FCAP,), jnp.int32),
            pltpu.VMEM((16,), jnp.int32),
        ],
        compiler_params=pltpu.CompilerParams(needs_layout_passes=False),
    )
    return k(e_src_flat, e_dst_flat, maski)


def _scales_body(degp_ref, maskf_ref, ssrc_ref, sdst_ref):
    deg = jnp.sum(degp_ref[...], axis=0)
    m = maskf_ref[...][:, 0][None, :]
    s = lax.rsqrt(jnp.maximum(deg, 1.0)) * m
    ssrc_ref[...] = s[0][:, None]
    sdst_ref[...] = s[1][:, None]


def _scales(degp, maskf):
    return pl.pallas_call(
        _scales_body,
        out_shape=[
            jax.ShapeDtypeStruct((NPAD, 1), jnp.float32),
            jax.ShapeDtypeStruct((NPAD, 1), jnp.float32),
        ],
    )(degp, maskf)


def _mul_body(h_ref, s_ref, g0_ref, g1_ref):
    g = h_ref[...] * s_ref[...]
    g0_ref[...] = g[:, :DH]
    g1_ref[...] = g[:, DH:]


def _prescale(h, ssrc):
    rows = 400
    grid = N // rows
    return pl.pallas_call(
        _mul_body,
        grid=(grid,),
        in_specs=[
            pl.BlockSpec((rows, D), lambda i: (i, 0)),
            pl.BlockSpec((rows, 1), lambda i: (i, 0)),
        ],
        out_specs=[
            pl.BlockSpec((rows, DH), lambda i: (i, 0)),
            pl.BlockSpec((rows, DH), lambda i: (i, 0)),
        ],
        out_shape=[
            jax.ShapeDtypeStruct((N, DH), jnp.float32),
            jax.ShapeDtypeStruct((N, DH), jnp.float32),
        ],
    )(h, ssrc)


DH = D // 2
FC = 80
NFC = FCAP // FC


def _agg_body(g0, g1, e_src, e_dst, cnt, out0, out1,
              srcv, dstv, cntv, bufa, bufb, bounce, a_sh, sem_a, sem_b):
    c = lax.axis_index("c")
    s = lax.axis_index("s")

    zeros16 = jnp.zeros((16,), jnp.float32)

    def zero_body(i, _):
        for kk in range(DH // 16):
            bounce[i, pl.ds(kk * 16, 16)] = zeros16
        return _

    pltpu.sync_copy(e_src.at[c, s], srcv)
    pltpu.sync_copy(e_dst.at[c, s], dstv)
    pltpu.sync_copy(cnt.at[c, s], cntv)
    nch = lax.reduce_max(cntv[0, pl.ds(0, 16)], (0,))
    nch = jnp.minimum(jnp.maximum(nch, 0), NFC)

    for g_hbm, out in ((g0, out0), (g1, out1)):
        lax.fori_loop(0, 128, zero_body, None)
        for kk in range(8):
            pltpu.sync_copy(bounce.at[pl.ds(0, 80)],
                            a_sh.at[pl.ds(s * 640 + kk * 80, 80)])
        plsc.subcore_barrier()

        @pl.when(nch > 0)
        def _():
            pltpu.async_copy(g_hbm.at[srcv.at[0]], bufa, sem_a)

        def edge_pair(j, _):
            a = 2 * j
            pltpu.make_async_copy(g_hbm.at[srcv.at[0]], bufa, sem_a).wait()
            pltpu.async_copy(g_hbm.at[srcv.at[a + 1]], bufb, sem_b)
            pltpu.sync_copy(bufa, a_sh.at[dstv.at[a]], add=True)
            pltpu.make_async_copy(g_hbm.at[srcv.at[0]], bufb, sem_b).wait()

            @pl.when(a + 2 < nch)
            def _():
                pltpu.async_copy(g_hbm.at[srcv.at[a + 2]], bufa, sem_a)

            pltpu.sync_copy(bufb, a_sh.at[dstv.at[a + 1]], add=True)
            return _

        lax.fori_loop(0, nch // 2, edge_pair, None)
        plsc.subcore_barrier()

        for kk in range(5):
            base = s * 640 + kk * 128
            pltpu.sync_copy(a_sh.at[pl.ds(base, 128)], bounce)
            pltpu.sync_copy(bounce, out.at[c, pl.ds(base, 128)])
        plsc.subcore_barrier()


def _aggregate(g0, g1, e_src_chunk, e_dst_chunk, cnt_chunk):
    k = pl.kernel(
        _agg_body,
        out_type=[
            jax.ShapeDtypeStruct((NC, NPAD, DH), jnp.float32),
            jax.ShapeDtypeStruct((NC, NPAD, DH), jnp.float32),
        ],
        mesh=_sc_mesh(),
        scratch_types=[
            pltpu.VMEM((NFC, FC), jnp.int32),
            pltpu.VMEM((NFC, FC), jnp.int32),
            pltpu.VMEM((1, 16), jnp.int32),
            pltpu.VMEM((FC, DH), jnp.float32),
            pltpu.VMEM((FC, DH), jnp.float32),
            pltpu.VMEM((128, DH), jnp.float32),
            pltpu.VMEM_SHARED((NPAD, DH), jnp.float32),
            pltpu.SemaphoreType.DMA,
            pltpu.SemaphoreType.DMA,
        ],
        compiler_params=pltpu.CompilerParams(
            use_tc_tiling_on_sc=False, needs_layout_passes=False),
    )
    return k(g0, g1, e_src_chunk, e_dst_chunk, cnt_chunk)


def _out_body(p0_ref, p1_ref, sdst_ref, w0_ref, w1_ref, b_ref, o_ref):
    sd = sdst_ref[...]
    a0 = (p0_ref[0] + p0_ref[1]) * sd
    a1 = (p1_ref[0] + p1_ref[1]) * sd
    o_ref[...] = (
        jnp.dot(a0, w0_ref[...], preferred_element_type=jnp.float32)
        + jnp.dot(a1, w1_ref[...], preferred_element_type=jnp.float32)
        + b_ref[...]
    )


def _finalize(p0, p1, sdst, W, b2):
    rows = 400
    grid = N // rows
    return pl.pallas_call(
        _out_body,
        grid=(grid,),
        in_specs=[
            pl.BlockSpec((NC, rows, DH), lambda i: (0, i, 0)),
            pl.BlockSpec((NC, rows, DH), lambda i: (0, i, 0)),
            pl.BlockSpec((rows, 1), lambda i: (i, 0)),
            pl.BlockSpec((DH, D), lambda i: (0, 0)),
            pl.BlockSpec((DH, D), lambda i: (0, 0)),
            pl.BlockSpec((1, D), lambda i: (0, 0)),
        ],
        out_specs=pl.BlockSpec((rows, D), lambda i: (i, 0)),
        out_shape=jax.ShapeDtypeStruct((N, D), jnp.float32),
    )(p0, p1, sdst, W[:DH], W[DH:], b2)


def kernel(h, mask, edge_index, W, b):
    src = edge_index[0]
    dst = edge_index[1]
    e_src_flat = src.reshape(NC, NS, EPW)
    e_dst_flat = dst.reshape(NC, NS, EPW)
    maski = jnp.pad(mask.astype(jnp.int32), (0, NPAD - N))
    maskf = jnp.pad(mask.astype(jnp.float32), (0, NPAD - N)).reshape(NPAD, 1)

    degp, fidx, cnt = _degrees(e_src_flat, e_dst_flat, maski)
    ssrc, sdst = _scales(degp, maskf)
    g0, g1 = _prescale(h, ssrc)
    p0, p1 = _aggregate(g0, g1,
                        fidx[:, 0].reshape(NC, NS, NFC, FC),
                        fidx[:, 1].reshape(NC, NS, NFC, FC),
                        cnt.reshape(NC, NS, 1, 16))
    return _finalize(p0, p1, sdst, W, b.reshape(1, D))


# speedup vs baseline: 35.2991x; 1.0297x over previous
import jax
import jax.numpy as jnp
from jax import lax
from jax.experimental import pallas as pl
from jax.experimental.pallas import tpu as pltpu
from jax.experimental.pallas import tpu_sc as plsc

N = 10000
E = 320000
D = 128

NC = 2
NS = 16
NW = NC * NS
EPW = E // NW
NPAD = 10240


def _sc_mesh():
    return plsc.VectorSubcoreMesh(core_axis_name="c", subcore_axis_name="s")


FCAP = 10080


def _deg_body(e_src, e_dst, maski, out, fidx, cnt,
              srcv, dstv, maskv, deg_s, deg_d, fsrcv, fdstv, cntv):
    c = lax.axis_index("c")
    s = lax.axis_index("s")
    wid = c * NS + s
    pltpu.sync_copy(e_src.at[c, s], srcv)
    pltpu.sync_copy(e_dst.at[c, s], dstv)
    pltpu.sync_copy(maski, maskv)

    zeros16 = jnp.zeros((16,), jnp.float32)
    iota16 = lax.iota(jnp.int32, 16)

    def zero_body(i, _):
        deg_s[pl.ds(i * 16, 16)] = zeros16
        deg_d[pl.ds(i * 16, 16)] = zeros16
        return _

    lax.fori_loop(0, NPAD // 16, zero_body, None)

    def fill_body(i, _):
        base = i * 16
        fsrcv[pl.ds(base, 16)] = ((wid * 613 + base) % 9984) + iota16
        fdstv[pl.ds(base, 16)] = N + ((base + wid) % 224) + iota16
        return _

    lax.fori_loop(0, FCAP // 16, fill_body, None)

    ones16 = jnp.ones((16,), jnp.float32)

    def hist_body(i, off):
        vs = srcv[pl.ds(i * 16, 16)]
        vd = dstv[pl.ds(i * 16, 16)]
        plsc.addupdate_scatter(deg_s, [vs], ones16)
        plsc.addupdate_scatter(deg_d, [vd], ones16)
        ms = plsc.load_gather(maskv, [vs])
        md = plsc.load_gather(maskv, [vd])
        act = (ms & md) != 0
        plsc.store_compressed(fsrcv.at[pl.ds(off, 16)], vs, mask=act)
        plsc.store_compressed(fdstv.at[pl.ds(off, 16)], vd, mask=act)
        npc = plsc.all_reduce_population_count(act)
        return off + lax.reduce_max(npc, (0,))

    off = lax.fori_loop(0, EPW // 16, hist_body, 0)

    end = ((off + 2 * FC - 1) // (2 * FC)) * (2 * FC)

    def pad_body(i, _):
        base = off + i * 16
        fsrcv[pl.ds(base, 16)] = ((wid * 613 + base) % 9984) + iota16
        fdstv[pl.ds(base, 16)] = N + ((base + wid) % 224) + iota16
        return _

    lax.fori_loop(0, (end - off) // 16, pad_body, None)

    cntv[pl.ds(0, 16)] = jnp.broadcast_to(end // FC, (16,))

    plsc.subcore_barrier()

    pltpu.sync_copy(deg_s, out.at[wid, 0])
    pltpu.sync_copy(deg_d, out.at[wid, 1])
    pltpu.sync_copy(fsrcv, fidx.at[wid, 0])
    pltpu.sync_copy(fdstv, fidx.at[wid, 1])
    pltpu.sync_copy(cntv, cnt.at[wid, 0])


def _degrees(e_src_flat, e_dst_flat, maski):
    k = pl.kernel(
        _deg_body,
        out_type=[
            jax.ShapeDtypeStruct((NW, 2, NPAD), jnp.float32),
            jax.ShapeDtypeStruct((NW, 2, FCAP), jnp.int32),
            jax.ShapeDtypeStruct((NW, 1, 16), jnp.int32),
        ],
        mesh=_sc_mesh(),
        scratch_types=[
            pltpu.VMEM((EPW,), jnp.int32),
            pltpu.VMEM((EPW,), jnp.int32),
            pltpu.VMEM((NPAD,), jnp.int32),
            pltpu.VMEM((NPAD,), jnp.float32),
            pltpu.VMEM((NPAD,), jnp.float32),
            pltpu.VMEM((FCAP,), jnp.int32),
            pltpu.VMEM((FCAP,), jnp.int32),
            pltpu.VMEM((16,), jnp.int32),
        ],
        compiler_params=pltpu.CompilerParams(needs_layout_passes=False),
    )
    return k(e_src_flat, e_dst_flat, maski)


def _scale_mul_body(degp_ref, maskf_ref, h_ref, g0_ref, g1_ref, sdst_ref):
    deg = jnp.sum(degp_ref[...], axis=0)
    m = maskf_ref[...][:, 0][None, :]
    s = lax.rsqrt(jnp.maximum(deg, 1.0)) * m
    g = h_ref[...] * s[0][:, None]
    g0_ref[...] = g[:, :DH]
    g1_ref[...] = g[:, DH:]
    sdst_ref[...] = s[1][:, None]


def _prescale(degp, maskf, h_pad):
    rows = 512
    grid = NPAD // rows
    return pl.pallas_call(
        _scale_mul_body,
        grid=(grid,),
        in_specs=[
            pl.BlockSpec((NW, 2, rows), lambda i: (0, 0, i)),
            pl.BlockSpec((rows, 1), lambda i: (i, 0)),
            pl.BlockSpec((rows, D), lambda i: (i, 0)),
        ],
        out_specs=[
            pl.BlockSpec((rows, DH), lambda i: (i, 0)),
            pl.BlockSpec((rows, DH), lambda i: (i, 0)),
            pl.BlockSpec((rows, 1), lambda i: (i, 0)),
        ],
        out_shape=[
            jax.ShapeDtypeStruct((NPAD, DH), jnp.float32),
            jax.ShapeDtypeStruct((NPAD, DH), jnp.float32),
            jax.ShapeDtypeStruct((NPAD, 1), jnp.float32),
        ],
    )(degp, maskf, h_pad)


DH = D // 2
FC = 80
NFC = FCAP // FC


def _agg_body(g0, g1, e_src, e_dst, cnt, out0, out1,
              srcv, dstv, cntv, bufa, bufb, bounce, a_sh, sem_a, sem_b):
    c = lax.axis_index("c")
    s = lax.axis_index("s")

    zeros16 = jnp.zeros((16,), jnp.float32)

    def zero_body(i, _):
        for kk in range(DH // 16):
            bounce[i, pl.ds(kk * 16, 16)] = zeros16
        return _

    pltpu.sync_copy(e_src.at[c, s], srcv)
    pltpu.sync_copy(e_dst.at[c, s], dstv)
    pltpu.sync_copy(cnt.at[c, s], cntv)
    nch = lax.reduce_max(cntv[0, pl.ds(0, 16)], (0,))
    nch = jnp.minimum(jnp.maximum(nch, 0), NFC)

    for g_hbm, out in ((g0, out0), (g1, out1)):
        lax.fori_loop(0, 128, zero_body, None)
        for kk in range(8):
            pltpu.sync_copy(bounce.at[pl.ds(0, 80)],
                            a_sh.at[pl.ds(s * 640 + kk * 80, 80)])
        plsc.subcore_barrier()

        @pl.when(nch > 0)
        def _():
            pltpu.async_copy(g_hbm.at[srcv.at[0]], bufa, sem_a)

        def edge_pair(j, _):
            a = 2 * j
            pltpu.make_async_copy(g_hbm.at[srcv.at[0]], bufa, sem_a).wait()
            pltpu.async_copy(g_hbm.at[srcv.at[a + 1]], bufb, sem_b)
            pltpu.sync_copy(bufa, a_sh.at[dstv.at[a]], add=True)
            pltpu.make_async_copy(g_hbm.at[srcv.at[0]], bufb, sem_b).wait()

            @pl.when(a + 2 < nch)
            def _():
                pltpu.async_copy(g_hbm.at[srcv.at[a + 2]], bufa, sem_a)

            pltpu.sync_copy(bufb, a_sh.at[dstv.at[a + 1]], add=True)
            return _

        lax.fori_loop(0, nch // 2, edge_pair, None)
        plsc.subcore_barrier()

        for kk in range(5):
            base = s * 640 + kk * 128
            pltpu.sync_copy(a_sh.at[pl.ds(base, 128)], bounce)
            pltpu.sync_copy(bounce, out.at[c, pl.ds(base, 128)])
        plsc.subcore_barrier()


def _aggregate(g0, g1, e_src_chunk, e_dst_chunk, cnt_chunk):
    k = pl.kernel(
        _agg_body,
        out_type=[
            jax.ShapeDtypeStruct((NC, NPAD, DH), jnp.float32),
            jax.ShapeDtypeStruct((NC, NPAD, DH), jnp.float32),
        ],
        mesh=_sc_mesh(),
        scratch_types=[
            pltpu.VMEM((NFC, FC), jnp.int32),
            pltpu.VMEM((NFC, FC), jnp.int32),
            pltpu.VMEM((1, 16), jnp.int32),
            pltpu.VMEM((FC, DH), jnp.float32),
            pltpu.VMEM((FC, DH), jnp.float32),
            pltpu.VMEM((128, DH), jnp.float32),
            pltpu.VMEM_SHARED((NPAD, DH), jnp.float32),
            pltpu.SemaphoreType.DMA,
            pltpu.SemaphoreType.DMA,
        ],
        compiler_params=pltpu.CompilerParams(
            use_tc_tiling_on_sc=False, needs_layout_passes=False),
    )
    return k(g0, g1, e_src_chunk, e_dst_chunk, cnt_chunk)


def _out_body(p0_ref, p1_ref, sdst_ref, w0_ref, w1_ref, b_ref, o_ref):
    sd = sdst_ref[...]
    a0 = (p0_ref[0] + p0_ref[1]) * sd
    a1 = (p1_ref[0] + p1_ref[1]) * sd
    o_ref[...] = (
        jnp.dot(a0, w0_ref[...], preferred_element_type=jnp.float32)
        + jnp.dot(a1, w1_ref[...], preferred_element_type=jnp.float32)
        + b_ref[...]
    )


def _finalize(p0, p1, sdst, W, b2):
    rows = 400
    grid = N // rows
    return pl.pallas_call(
        _out_body,
        grid=(grid,),
        in_specs=[
            pl.BlockSpec((NC, rows, DH), lambda i: (0, i, 0)),
            pl.BlockSpec((NC, rows, DH), lambda i: (0, i, 0)),
            pl.BlockSpec((rows, 1), lambda i: (i, 0)),
            pl.BlockSpec((DH, D), lambda i: (0, 0)),
            pl.BlockSpec((DH, D), lambda i: (0, 0)),
            pl.BlockSpec((1, D), lambda i: (0, 0)),
        ],
        out_specs=pl.BlockSpec((rows, D), lambda i: (i, 0)),
        out_shape=jax.ShapeDtypeStruct((N, D), jnp.float32),
    )(p0, p1, sdst, W[:DH], W[DH:], b2)


def kernel(h, mask, edge_index, W, b):
    src = edge_index[0]
    dst = edge_index[1]
    e_src_flat = src.reshape(NC, NS, EPW)
    e_dst_flat = dst.reshape(NC, NS, EPW)
    maski = jnp.pad(mask.astype(jnp.int32), (0, NPAD - N))
    maskf = jnp.pad(mask.astype(jnp.float32), (0, NPAD - N)).reshape(NPAD, 1)

    h_pad = jnp.pad(h, ((0, NPAD - N), (0, 0)))

    degp, fidx, cnt = _degrees(e_src_flat, e_dst_flat, maski)
    g0, g1, sdst = _prescale(degp, maskf, h_pad)
    p0, p1 = _aggregate(g0, g1,
                        fidx[:, 0].reshape(NC, NS, NFC, FC),
                        fidx[:, 1].reshape(NC, NS, NFC, FC),
                        cnt.reshape(NC, NS, 1, 16))
    return _finalize(p0, p1, sdst, W, b.reshape(1, D))
